# baseline (speedup 1.0000x reference)
import functools

import jax
import jax.numpy as jnp
from jax import lax
from jax.experimental import pallas as pl
from jax.experimental.pallas import tpu as pltpu
from jax.experimental.pallas import tpu_sc as plsc

BATCH = 16384
DIM = 64
_INFO = plsc.get_sparse_core_info()
NC, NS, NL = _INFO.num_cores, _INFO.num_subcores, _INFO.num_lanes
NW = NC * NS
B_PER_W = BATCH // NW
IDX_CHUNK = 128
N_CHUNKS = B_PER_W // IDX_CHUNK


def _sc_body(table_hbm, idx_hbm, out_hbm, idx_v, rows_v, out_v, sem):
    wid = lax.axis_index("s") * NC + lax.axis_index("c")
    base = wid * B_PER_W

    for j in range(N_CHUNKS):
        pltpu.sync_copy(idx_hbm.at[pl.ds(base + j * IDX_CHUNK, IDX_CHUNK)],
                        idx_v.at[j])

    copies = []
    for j in range(N_CHUNKS):
        copies.append(pltpu.async_copy(
            table_hbm.at[idx_v.at[j]],
            rows_v.at[pl.ds(j * IDX_CHUNK, IDX_CHUNK)],
            sem))
    for c in copies:
        c.wait()

    lane_iota = lax.iota(jnp.int32, NL)

    def _swap(x, d):
        perm = lane_iota ^ d
        return x.at[perm].get(mode="promise_in_bounds")

    def group_body(g, carry):
        vs = []
        for j in range(NL):
            row = rows_v.at[g * NL + j]
            a = row[pl.ds(0, NL)] + row[pl.ds(NL, NL)]
            b = row[pl.ds(2 * NL, NL)] + row[pl.ds(3 * NL, NL)]
            vs.append(a + b)
        d = 1
        while len(vs) > 1:
            nxt = []
            for k in range(0, len(vs), 2):
                a2 = vs[k] + _swap(vs[k], d)
                b2 = vs[k + 1] + _swap(vs[k + 1], d)
                nxt.append(jnp.where((lane_iota & d) == 0, a2, b2))
            vs = nxt
            d *= 2
        out_v[pl.ds(g * NL, NL)] = -vs[0]
        return carry

    lax.fori_loop(0, B_PER_W // NL, group_body, 0)

    pltpu.sync_copy(out_v, out_hbm.at[pl.ds(base, B_PER_W)])


@jax.jit
def _score(ent_embeddings, batch_h):
    mesh = plsc.VectorSubcoreMesh(core_axis_name="c", subcore_axis_name="s")
    run = pl.kernel(
        _sc_body,
        out_type=jax.ShapeDtypeStruct((BATCH,), jnp.float32),
        mesh=mesh,
        scratch_types=[
            pltpu.VMEM((N_CHUNKS, IDX_CHUNK), jnp.int32),
            pltpu.VMEM((B_PER_W, DIM), jnp.float32),
            pltpu.VMEM((B_PER_W,), jnp.float32),
            pltpu.SemaphoreType.DMA,
        ],
        compiler_params=pltpu.CompilerParams(use_tc_tiling_on_sc=False),
    )
    return run(ent_embeddings, batch_h)


def kernel(ent_embeddings, rel_matrices, batch_h, batch_t, batch_r):
    return _score(ent_embeddings, batch_h)


# speedup vs baseline: 4.5589x; 4.5589x over previous
import jax
import jax.numpy as jnp
from jax import lax
from jax.experimental import pallas as pl
from jax.experimental.pallas import tpu as pltpu
from jax.experimental.pallas import tpu_sc as plsc

ENT = 1_000_000
BATCH = 16384
DIM = 64
_INFO = plsc.get_sparse_core_info()
NC, NS, NL = _INFO.num_cores, _INFO.num_subcores, _INFO.num_lanes
NW = NC * NS
B_PER_W = BATCH // NW
IDX_CHUNK = 128
N_CHUNKS = B_PER_W // IDX_CHUNK

COLSUM_BLOCK = 8192


def _colsum_body(x_ref, o_ref):
    o_ref[...] = -jnp.sum(x_ref[...], axis=0)


def _gather_body(colsum_hbm, idx_hbm, out_hbm, idx_v, vals_v, sem):
    wid = lax.axis_index("s") * NC + lax.axis_index("c")
    base = wid * B_PER_W

    for j in range(N_CHUNKS):
        pltpu.sync_copy(idx_hbm.at[pl.ds(base + j * IDX_CHUNK, IDX_CHUNK)],
                        idx_v.at[j])

    copies = []
    for j in range(N_CHUNKS):
        copies.append(pltpu.async_copy(
            colsum_hbm.at[idx_v.at[j]],
            vals_v.at[pl.ds(j * IDX_CHUNK, IDX_CHUNK)],
            sem))
    for c in copies:
        c.wait()

    pltpu.sync_copy(vals_v, out_hbm.at[pl.ds(base, B_PER_W)])


@jax.jit
def _score(ent_embeddings, batch_h):
    table_t = ent_embeddings.T

    colsum = pl.pallas_call(
        _colsum_body,
        out_shape=jax.ShapeDtypeStruct((ENT,), jnp.float32),
        grid=(pl.cdiv(ENT, COLSUM_BLOCK),),
        in_specs=[pl.BlockSpec((DIM, COLSUM_BLOCK), lambda i: (0, i))],
        out_specs=pl.BlockSpec((COLSUM_BLOCK,), lambda i: (i,)),
    )(table_t)

    mesh = plsc.VectorSubcoreMesh(core_axis_name="c", subcore_axis_name="s")
    run = pl.kernel(
        _gather_body,
        out_type=jax.ShapeDtypeStruct((BATCH,), jnp.float32),
        mesh=mesh,
        scratch_types=[
            pltpu.VMEM((N_CHUNKS, IDX_CHUNK), jnp.int32),
            pltpu.VMEM((B_PER_W,), jnp.float32),
            pltpu.SemaphoreType.DMA,
        ],
    )
    return run(colsum, batch_h)


def kernel(ent_embeddings, rel_matrices, batch_h, batch_t, batch_r):
    return _score(ent_embeddings, batch_h)


# speedup vs baseline: 6.4260x; 1.4096x over previous
import jax
import jax.numpy as jnp
from jax import lax
from jax.experimental import pallas as pl
from jax.experimental.pallas import tpu as pltpu
from jax.experimental.pallas import tpu_sc as plsc

ENT = 1_000_000
BATCH = 16384
DIM = 64
_INFO = plsc.get_sparse_core_info()
NC, NS, NL = _INFO.num_cores, _INFO.num_subcores, _INFO.num_lanes
NW = NC * NS
B_PER_W = BATCH // NW
IDX_CHUNK = 128
N_CHUNKS = B_PER_W // IDX_CHUNK

COLSUM_BLOCK = 32768


def _colsum_body(x_ref, o_ref):
    o_ref[...] = -jnp.sum(x_ref[...], axis=0)


def _gather_body(colsum_hbm, idx_hbm, out_hbm, idx_v, vals_v, sem):
    wid = lax.axis_index("s") * NC + lax.axis_index("c")
    base = wid * B_PER_W

    for j in range(N_CHUNKS):
        pltpu.sync_copy(idx_hbm.at[pl.ds(base + j * IDX_CHUNK, IDX_CHUNK)],
                        idx_v.at[j])

    copies = []
    for j in range(N_CHUNKS):
        copies.append(pltpu.async_copy(
            colsum_hbm.at[idx_v.at[j]],
            vals_v.at[pl.ds(j * IDX_CHUNK, IDX_CHUNK)],
            sem))
    for c in copies:
        c.wait()

    pltpu.sync_copy(vals_v, out_hbm.at[pl.ds(base, B_PER_W)])


@jax.jit
def _score(ent_embeddings, batch_h):
    table_t = ent_embeddings.T

    colsum = pl.pallas_call(
        _colsum_body,
        out_shape=jax.ShapeDtypeStruct((ENT,), jnp.float32),
        grid=(pl.cdiv(ENT, COLSUM_BLOCK),),
        in_specs=[pl.BlockSpec((DIM, COLSUM_BLOCK), lambda i: (0, i))],
        out_specs=pl.BlockSpec((COLSUM_BLOCK,), lambda i: (i,)),
    )(table_t)

    mesh = plsc.VectorSubcoreMesh(core_axis_name="c", subcore_axis_name="s")
    run = pl.kernel(
        _gather_body,
        out_type=jax.ShapeDtypeStruct((BATCH,), jnp.float32),
        mesh=mesh,
        scratch_types=[
            pltpu.VMEM((N_CHUNKS, IDX_CHUNK), jnp.int32),
            pltpu.VMEM((B_PER_W,), jnp.float32),
            pltpu.SemaphoreType.DMA,
        ],
    )
    return run(colsum, batch_h)


def kernel(ent_embeddings, rel_matrices, batch_h, batch_t, batch_r):
    return _score(ent_embeddings, batch_h)
